# Initial kernel scaffold; baseline (speedup 1.0000x reference)
import jax
import jax.numpy as jnp
from jax.experimental import pallas as pl


def kernel(x, W_enc, b_enc, W_dec, b_dec):
    raise NotImplementedError("write your pallas kernel here")


# speedup vs baseline: 2.5210x; 2.5210x over previous
import functools

import jax
import jax.numpy as jnp
from jax.experimental import pallas as pl
from jax.experimental.pallas import tpu as pltpu

D_IN = 2048
N_LAT = 16384
TOPK = 64
NB = 4096

NEG_INF = float("-inf")


def _enc_body(x_ref, w_ref, b_ref, o_ref):
    o_ref[...] = (
        jnp.dot(x_ref[...], w_ref[...], preferred_element_type=jnp.float32)
        + b_ref[...]
    )


def _encode(x, w_enc, b_enc2d):
    bm, bn = 256, 1024
    grid = (NB // bm, N_LAT // bn)
    return pl.pallas_call(
        _enc_body,
        grid=grid,
        in_specs=[
            pl.BlockSpec((bm, D_IN), lambda i, j: (i, 0)),
            pl.BlockSpec((D_IN, bn), lambda i, j: (0, j)),
            pl.BlockSpec((1, bn), lambda i, j: (0, j)),
        ],
        out_specs=pl.BlockSpec((bm, bn), lambda i, j: (i, j)),
        out_shape=jax.ShapeDtypeStruct((NB, N_LAT), jnp.float32),
        compiler_params=pltpu.CompilerParams(
            dimension_semantics=("parallel", "parallel"),
        ),
    )(x, w_enc, b_enc2d)


def _topk_body(pre_ref, idx_ref, acts_ref, work_ref, valsT_ref, idxT_ref):
    bm = pre_ref.shape[0]
    work_ref[...] = pre_ref[...]
    col = jax.lax.broadcasted_iota(jnp.int32, (bm, N_LAT), 1)

    def step(j, _):
        p = work_ref[...]
        v = jnp.max(p, axis=1)
        i = jnp.argmax(p, axis=1).astype(jnp.int32)
        valsT_ref[pl.ds(j, 1), :] = v[None, :]
        idxT_ref[pl.ds(j, 1), :] = i[None, :]
        work_ref[...] = jnp.where(col == i[:, None], NEG_INF, p)
        return 0

    jax.lax.fori_loop(0, TOPK, step, 0)
    idx_ref[...] = idxT_ref[...].T
    v64 = valsT_ref[TOPK - 1, :][:, None]
    pre = pre_ref[...]
    acts_ref[...] = jnp.where(pre >= v64, jnp.maximum(pre, 0.0), 0.0)


def _topk(pre):
    bm = 64
    grid = (NB // bm,)
    return pl.pallas_call(
        _topk_body,
        grid=grid,
        in_specs=[pl.BlockSpec((bm, N_LAT), lambda i: (i, 0))],
        out_specs=[
            pl.BlockSpec((bm, TOPK), lambda i: (i, 0)),
            pl.BlockSpec((bm, N_LAT), lambda i: (i, 0)),
        ],
        out_shape=[
            jax.ShapeDtypeStruct((NB, TOPK), jnp.int32),
            jax.ShapeDtypeStruct((NB, N_LAT), jnp.float32),
        ],
        scratch_shapes=[
            pltpu.VMEM((bm, N_LAT), jnp.float32),
            pltpu.VMEM((TOPK, bm), jnp.float32),
            pltpu.VMEM((TOPK, bm), jnp.int32),
        ],
        compiler_params=pltpu.CompilerParams(
            dimension_semantics=("parallel",),
        ),
    )(pre)


def _dec_body(a_ref, w_ref, b_ref, o_ref):
    k = pl.program_id(1)

    @pl.when(k == 0)
    def _():
        o_ref[...] = jnp.broadcast_to(b_ref[...], o_ref.shape)

    o_ref[...] += jnp.dot(
        a_ref[...], w_ref[...], preferred_element_type=jnp.float32
    )


def _decode(acts, w_dec, b_dec2d):
    bm, bk = 512, 2048
    grid = (NB // bm, N_LAT // bk)
    return pl.pallas_call(
        _dec_body,
        grid=grid,
        in_specs=[
            pl.BlockSpec((bm, bk), lambda i, k: (i, k)),
            pl.BlockSpec((bk, D_IN), lambda i, k: (k, 0)),
            pl.BlockSpec((1, D_IN), lambda i, k: (0, 0)),
        ],
        out_specs=pl.BlockSpec((bm, D_IN), lambda i, k: (i, 0)),
        out_shape=jax.ShapeDtypeStruct((NB, D_IN), jnp.float32),
        compiler_params=pltpu.CompilerParams(
            dimension_semantics=("parallel", "arbitrary"),
        ),
    )(acts, w_dec, b_dec2d)


def kernel(x, W_enc, b_enc, W_dec, b_dec):
    pre = _encode(x, W_enc, b_enc.reshape(1, N_LAT))
    topk_idx, acts = _topk(pre)
    recon = _decode(acts, W_dec, b_dec.reshape(1, D_IN))
    return (recon, acts, topk_idx)


# speedup vs baseline: 6.8094x; 2.7010x over previous
import functools

import jax
import jax.numpy as jnp
from jax import lax
from jax.experimental import pallas as pl
from jax.experimental.pallas import tpu as pltpu
from jax.experimental.pallas import tpu_sc as plsc

D_IN = 2048
N_LAT = 16384
TOPK = 64
NB = 4096

CHUNK = 64
NCHUNK = N_LAT // CHUNK

NEG_INF = float("-inf")


def _enc_body(x_ref, w_ref, b_ref, o_ref, m_ref):
    bm, bn = o_ref.shape
    p = (
        jnp.dot(x_ref[...], w_ref[...], preferred_element_type=jnp.float32)
        + b_ref[...]
    )
    o_ref[...] = p
    m_ref[...] = jnp.max(p.reshape(bm, bn // CHUNK, CHUNK), axis=2)[None]


def _encode(x, w_enc, b_enc2d):
    bm, bn = 256, 1024
    grid = (NB // bm, N_LAT // bn)
    return pl.pallas_call(
        _enc_body,
        grid=grid,
        in_specs=[
            pl.BlockSpec((bm, D_IN), lambda i, j: (i, 0)),
            pl.BlockSpec((D_IN, bn), lambda i, j: (0, j)),
            pl.BlockSpec((1, bn), lambda i, j: (0, j)),
        ],
        out_specs=[
            pl.BlockSpec((bm, bn), lambda i, j: (i, j)),
            pl.BlockSpec((1, bm, bn // CHUNK), lambda i, j: (j, i, 0)),
        ],
        out_shape=[
            jax.ShapeDtypeStruct((NB, N_LAT), jnp.float32),
            jax.ShapeDtypeStruct((N_LAT // bn, NB, bn // CHUNK), jnp.float32),
        ],
        compiler_params=pltpu.CompilerParams(
            dimension_semantics=("parallel", "parallel"),
        ),
    )(x, w_enc, b_enc2d)


def _merge16(ak, ai, bk, bi):
    rbk = lax.rev(bk, (0,))
    rbi = lax.rev(bi, (0,))
    sel = ak >= rbk
    hk = jnp.where(sel, ak, rbk)
    hi_ = jnp.where(sel, ai, rbi)
    lk = jnp.where(sel, rbk, ak)
    li = jnp.where(sel, rbi, ai)
    hk, hi_ = plsc.sort_key_val(hk, hi_, descending=True)
    lk, li = plsc.sort_key_val(lk, li, descending=True)
    return hk, hi_, lk, li


def _insert16(best, nk, ni):
    out = []
    for q in range(4):
        bk, bi = best[q]
        hk, hi_, nk, ni = _merge16(bk, bi, nk, ni)
        out.append((hk, hi_))
    return out


def _sc_select(pre, m):
    info = plsc.get_sparse_core_info()
    nw = info.num_cores * info.num_subcores
    rows_per_w = NB // nw
    mesh = plsc.VectorSubcoreMesh(core_axis_name="c", subcore_axis_name="s")

    @functools.partial(
        pl.kernel,
        mesh=mesh,
        out_type=[
            jax.ShapeDtypeStruct((NB, TOPK), jnp.int32),
            jax.ShapeDtypeStruct((NB, TOPK), jnp.float32),
        ],
        scratch_types=[
            pltpu.VMEM((N_LAT,), jnp.float32),
            pltpu.VMEM((NCHUNK,), jnp.float32),
            pltpu.VMEM((N_LAT + 16,), jnp.float32),
            pltpu.VMEM((N_LAT + 16,), jnp.int32),
            pltpu.VMEM((TOPK,), jnp.int32),
            pltpu.VMEM((TOPK,), jnp.float32),
        ],
        compiler_params=pltpu.CompilerParams(needs_layout_passes=False),
    )
    def body(pre_hbm, m_hbm, idx_hbm, val_hbm, rowbuf, mrow, cv, ci, oi, ov):
        wid = lax.axis_index("s") * info.num_cores + lax.axis_index("c")
        base = wid * rows_per_w
        lane = lax.iota(jnp.int32, 16)
        neg = jnp.full((16,), NEG_INF, jnp.float32)
        zero_i = jnp.zeros((16,), jnp.int32)

        def per_row(r, _):
            row = base + r
            pltpu.sync_copy(m_hbm.at[row], mrow)
            pltpu.sync_copy(pre_hbm.at[row], rowbuf)

            best = [(neg, zero_i)] * 4
            for k in range(NCHUNK // 16):
                kv = mrow[pl.ds(k * 16, 16)]
                kk, ii = plsc.sort_key_val(kv, zero_i, descending=True)
                best = _insert16(best, kk, ii)
            thr_s = jnp.min(best[3][0])
            thrv = jnp.full((16,), thr_s, jnp.float32)

            def scan_step(j, pos):
                v = rowbuf[pl.ds(j * 16, 16)]
                iv = j * 16 + lane
                msk = v >= thrv
                plsc.store_compressed(cv.at[pl.ds(pos, 16)], v, mask=msk)
                plsc.store_compressed(ci.at[pl.ds(pos, 16)], iv, mask=msk)
                return pos + jnp.sum(jnp.where(msk, 1, 0))

            cnt = lax.fori_loop(0, N_LAT // 16, scan_step, 0)
            cv[pl.ds(cnt, 16)] = neg

            def m_step(j, best):
                v = cv[pl.ds(j * 16, 16)]
                i = ci[pl.ds(j * 16, 16)]
                v, i = plsc.sort_key_val(v, i, descending=True)
                return _insert16(best, v, i)

            nvr = (cnt + 15) // 16
            best = [(neg, zero_i)] * 4
            best = lax.fori_loop(0, nvr, m_step, best)

            for q in range(4):
                ov[pl.ds(q * 16, 16)] = best[q][0]
                oi[pl.ds(q * 16, 16)] = best[q][1]
            pltpu.sync_copy(oi, idx_hbm.at[row])
            pltpu.sync_copy(ov, val_hbm.at[row])
            return 0

        lax.fori_loop(0, rows_per_w, per_row, 0)

    return body(pre, m)


def _dec_body(p_ref, v_ref, w_ref, b_ref, o_ref, a_ref):
    k = pl.program_id(1)
    v64 = v_ref[:, TOPK - 1 :]
    p = p_ref[...]
    a = jnp.where(p >= v64, jnp.maximum(p, 0.0), 0.0)
    a_ref[...] = a

    @pl.when(k == 0)
    def _():
        o_ref[...] = jnp.broadcast_to(b_ref[...], o_ref.shape)

    o_ref[...] += jnp.dot(a, w_ref[...], preferred_element_type=jnp.float32)


def _decode(pre, vals, w_dec, b_dec2d):
    bm, bk = 512, 1024
    grid = (NB // bm, N_LAT // bk)
    return pl.pallas_call(
        _dec_body,
        grid=grid,
        in_specs=[
            pl.BlockSpec((bm, bk), lambda i, k: (i, k)),
            pl.BlockSpec((bm, TOPK), lambda i, k: (i, 0)),
            pl.BlockSpec((bk, D_IN), lambda i, k: (k, 0)),
            pl.BlockSpec((1, D_IN), lambda i, k: (0, 0)),
        ],
        out_specs=[
            pl.BlockSpec((bm, D_IN), lambda i, k: (i, 0)),
            pl.BlockSpec((bm, bk), lambda i, k: (i, k)),
        ],
        out_shape=[
            jax.ShapeDtypeStruct((NB, D_IN), jnp.float32),
            jax.ShapeDtypeStruct((NB, N_LAT), jnp.float32),
        ],
        compiler_params=pltpu.CompilerParams(
            dimension_semantics=("parallel", "arbitrary"),
        ),
    )(pre, vals, w_dec, b_dec2d)


def kernel(x, W_enc, b_enc, W_dec, b_dec):
    pre, m3 = _encode(x, W_enc, b_enc.reshape(1, N_LAT))
    m = m3.transpose(1, 0, 2).reshape(NB, NCHUNK)
    topk_idx, topk_vals = _sc_select(pre, m)
    recon, acts = _decode(pre, topk_vals, W_dec, b_dec.reshape(1, D_IN))
    return (recon, acts, topk_idx)


# speedup vs baseline: 7.0233x; 1.0314x over previous
import functools

import jax
import jax.numpy as jnp
from jax import lax
from jax.experimental import pallas as pl
from jax.experimental.pallas import tpu as pltpu
from jax.experimental.pallas import tpu_sc as plsc

D_IN = 2048
N_LAT = 16384
TOPK = 64
NB = 4096

CHUNK = 64
NCHUNK = N_LAT // CHUNK

NEG_INF = float("-inf")


def _enc_body(x_ref, w_ref, b_ref, o_ref, m_ref):
    bm, bn = o_ref.shape
    p = (
        jnp.dot(x_ref[...], w_ref[...], preferred_element_type=jnp.float32)
        + b_ref[...]
    )
    o_ref[...] = p
    m_ref[...] = jnp.max(p.reshape(bm, bn // CHUNK, CHUNK), axis=2)[None]


def _encode(x, w_enc, b_enc2d):
    bm, bn = 256, 1024
    grid = (NB // bm, N_LAT // bn)
    return pl.pallas_call(
        _enc_body,
        grid=grid,
        in_specs=[
            pl.BlockSpec((bm, D_IN), lambda i, j: (i, 0)),
            pl.BlockSpec((D_IN, bn), lambda i, j: (0, j)),
            pl.BlockSpec((1, bn), lambda i, j: (0, j)),
        ],
        out_specs=[
            pl.BlockSpec((bm, bn), lambda i, j: (i, j)),
            pl.BlockSpec((1, bm, bn // CHUNK), lambda i, j: (j, i, 0)),
        ],
        out_shape=[
            jax.ShapeDtypeStruct((NB, N_LAT), jnp.float32),
            jax.ShapeDtypeStruct((N_LAT // bn, NB, bn // CHUNK), jnp.float32),
        ],
        compiler_params=pltpu.CompilerParams(
            dimension_semantics=("parallel", "parallel"),
        ),
    )(x, w_enc, b_enc2d)


def _merge16(ak, ai, bk, bi):
    rbk = lax.rev(bk, (0,))
    rbi = lax.rev(bi, (0,))
    sel = ak >= rbk
    hk = jnp.where(sel, ak, rbk)
    hi_ = jnp.where(sel, ai, rbi)
    lk = jnp.where(sel, rbk, ak)
    li = jnp.where(sel, rbi, ai)
    hk, hi_ = plsc.sort_key_val(hk, hi_, descending=True)
    lk, li = plsc.sort_key_val(lk, li, descending=True)
    return hk, hi_, lk, li


def _insert16(best, nk, ni):
    out = []
    for q in range(4):
        bk, bi = best[q]
        hk, hi_, nk, ni = _merge16(bk, bi, nk, ni)
        out.append((hk, hi_))
    return out


def _sc_select(pre, m):
    info = plsc.get_sparse_core_info()
    nw = info.num_cores * info.num_subcores
    rows_per_w = NB // nw
    mesh = plsc.VectorSubcoreMesh(core_axis_name="c", subcore_axis_name="s")

    @functools.partial(
        pl.kernel,
        mesh=mesh,
        out_type=[
            jax.ShapeDtypeStruct((NB, TOPK), jnp.int32),
            jax.ShapeDtypeStruct((NB, TOPK), jnp.float32),
        ],
        scratch_types=[
            pltpu.VMEM((N_LAT,), jnp.float32),
            pltpu.VMEM((NCHUNK,), jnp.float32),
            pltpu.VMEM((N_LAT + 16,), jnp.float32),
            pltpu.VMEM((N_LAT + 16,), jnp.int32),
            pltpu.VMEM((TOPK,), jnp.int32),
            pltpu.VMEM((TOPK,), jnp.float32),
        ],
        compiler_params=pltpu.CompilerParams(needs_layout_passes=False),
    )
    def body(pre_hbm, m_hbm, idx_hbm, val_hbm, rowbuf, mrow, cv, ci, oi, ov):
        wid = lax.axis_index("s") * info.num_cores + lax.axis_index("c")
        base = wid * rows_per_w
        lane = lax.iota(jnp.int32, 16)
        neg = jnp.full((16,), NEG_INF, jnp.float32)
        zero_i = jnp.zeros((16,), jnp.int32)

        def per_row(r, _):
            row = base + r
            pltpu.sync_copy(m_hbm.at[row], mrow)
            pltpu.sync_copy(pre_hbm.at[row], rowbuf)

            best = [(neg, zero_i)] * 4
            for k in range(NCHUNK // 16):
                kv = mrow[pl.ds(k * 16, 16)]
                kk, ii = plsc.sort_key_val(kv, zero_i, descending=True)
                best = _insert16(best, kk, ii)
            thr_s = jnp.min(best[3][0])
            thrv = jnp.full((16,), thr_s, jnp.float32)

            def scan_step(j, pos):
                v = rowbuf[pl.ds(j * 16, 16)]
                iv = j * 16 + lane
                msk = v >= thrv
                plsc.store_compressed(cv.at[pl.ds(pos, 16)], v, mask=msk)
                plsc.store_compressed(ci.at[pl.ds(pos, 16)], iv, mask=msk)
                return pos + jnp.sum(jnp.where(msk, 1, 0))

            cnt = lax.fori_loop(0, N_LAT // 16, scan_step, 0)
            cv[pl.ds(cnt, 16)] = neg

            def m_step(j, best):
                v = cv[pl.ds(j * 16, 16)]
                i = ci[pl.ds(j * 16, 16)]
                v, i = plsc.sort_key_val(v, i, descending=True)
                return _insert16(best, v, i)

            nvr = (cnt + 15) // 16
            best = [(neg, zero_i)] * 4
            best = lax.fori_loop(0, nvr, m_step, best)

            for q in range(4):
                ov[pl.ds(q * 16, 16)] = best[q][0]
                oi[pl.ds(q * 16, 16)] = best[q][1]
            pltpu.sync_copy(oi, idx_hbm.at[row])
            pltpu.sync_copy(ov, val_hbm.at[row])
            return 0

        lax.fori_loop(0, rows_per_w, per_row, 0)

    return body(pre, m)


def _dec_body(p_ref, v_ref, w_ref, b_ref, o_ref, a_ref):
    k = pl.program_id(1)
    v64 = v_ref[:, TOPK - 1 :]
    p = p_ref[...]
    a = jnp.where(p >= v64, jnp.maximum(p, 0.0), 0.0)
    a_ref[...] = a

    @pl.when(k == 0)
    def _():
        o_ref[...] = jnp.broadcast_to(b_ref[...], o_ref.shape)

    o_ref[...] += jnp.dot(
        a.astype(jnp.bfloat16), w_ref[...], preferred_element_type=jnp.float32
    )


def _decode(pre, vals, w_dec, b_dec2d):
    bm, bk = 512, 1024
    grid = (NB // bm, N_LAT // bk)
    return pl.pallas_call(
        _dec_body,
        grid=grid,
        in_specs=[
            pl.BlockSpec((bm, bk), lambda i, k: (i, k)),
            pl.BlockSpec((bm, TOPK), lambda i, k: (i, 0)),
            pl.BlockSpec((bk, D_IN), lambda i, k: (k, 0)),
            pl.BlockSpec((1, D_IN), lambda i, k: (0, 0)),
        ],
        out_specs=[
            pl.BlockSpec((bm, D_IN), lambda i, k: (i, 0)),
            pl.BlockSpec((bm, bk), lambda i, k: (i, k)),
        ],
        out_shape=[
            jax.ShapeDtypeStruct((NB, D_IN), jnp.float32),
            jax.ShapeDtypeStruct((NB, N_LAT), jnp.float32),
        ],
        compiler_params=pltpu.CompilerParams(
            dimension_semantics=("parallel", "arbitrary"),
        ),
    )(pre, vals, w_dec.astype(jnp.bfloat16), b_dec2d)


def kernel(x, W_enc, b_enc, W_dec, b_dec):
    pre, m3 = _encode(x, W_enc, b_enc.reshape(1, N_LAT))
    m = m3.transpose(1, 0, 2).reshape(NB, NCHUNK)
    topk_idx, topk_vals = _sc_select(pre, m)
    recon, acts = _decode(pre, topk_vals, W_dec, b_dec.reshape(1, D_IN))
    return (recon, acts, topk_idx)
